# Initial kernel scaffold; baseline (speedup 1.0000x reference)
import jax
import jax.numpy as jnp
from jax.experimental import pallas as pl


def kernel(x, W1t, b1t, W2t, b2t, W1f, b1f, W2f, b2f):
    raise NotImplementedError("write your pallas kernel here")


# speedup vs baseline: 1.8862x; 1.8862x over previous
import functools

import jax
import jax.numpy as jnp
from jax.experimental import pallas as pl
from jax.experimental.pallas import tpu as pltpu

N = 8192
D = 1024
F = 1024
THRESHOLD = 0.8
BLOCK_M = 512


def _fused_kernel(x_ref, w1t_ref, b1t_ref, w2t_ref, b2t_ref,
                  w1f_ref, b1f_ref, w2f_ref, b2f_ref, out_ref):
    x = x_ref[...]
    ht = jnp.maximum(
        jnp.dot(x, w1t_ref[...], preferred_element_type=jnp.float32)
        + b1t_ref[...], 0.0)
    t_out = (jnp.dot(ht, w2t_ref[...], preferred_element_type=jnp.float32)
             + b2t_ref[...])
    m = jnp.max(t_out, axis=1, keepdims=True)
    s = jnp.sum(jnp.exp(t_out - m), axis=1, keepdims=True)
    low_conf = 1.0 < THRESHOLD * s
    hf = jnp.maximum(
        jnp.dot(x, w1f_ref[...], preferred_element_type=jnp.float32)
        + b1f_ref[...], 0.0)
    f_out = (jnp.dot(hf, w2f_ref[...], preferred_element_type=jnp.float32)
             + b2f_ref[...])
    out_ref[...] = jnp.where(low_conf, f_out, t_out)


@jax.jit
def kernel(x, W1t, b1t, W2t, b2t, W1f, b1f, W2f, b2f):
    grid = (N // BLOCK_M,)
    row_spec = pl.BlockSpec((BLOCK_M, D), lambda i: (i, 0))
    w_spec = pl.BlockSpec((D, F), lambda i: (0, 0))
    b_spec = pl.BlockSpec((1, F), lambda i: (0, 0))
    out = pl.pallas_call(
        _fused_kernel,
        grid=grid,
        in_specs=[row_spec,
                  w_spec, b_spec, w_spec, b_spec,
                  w_spec, b_spec, w_spec, b_spec],
        out_specs=row_spec,
        out_shape=jax.ShapeDtypeStruct((N, D), jnp.float32),
        compiler_params=pltpu.CompilerParams(
            dimension_semantics=("arbitrary",),
        ),
    )(x, W1t, b1t.reshape(1, F), W2t, b2t.reshape(1, D),
      W1f, b1f.reshape(1, F), W2f, b2f.reshape(1, D))
    return out
